# baseline (speedup 1.0000x reference)
import functools

import jax
import jax.numpy as jnp
from jax import lax
from jax.experimental import pallas as pl
from jax.experimental.pallas import tpu as pltpu
from jax.experimental.pallas import tpu_sc as plsc

_B = 64
_R_TC = 768
_VOCAB = 1000
_EMB = 64
_SEQ = 50
_NC = 2
_NS = 16
_NW = _NC * _NS


def _argmax_embed_kernel(x_ref, dict_ref, out_ref):
    xb = x_ref[...]
    iota = lax.broadcasted_iota(jnp.int32, xb.shape, 2)
    m = jnp.max(xb, axis=-1, keepdims=True)
    eq = xb == m
    idx = jnp.min(jnp.where(eq, iota, _VOCAB), axis=-1, keepdims=True)
    onehot = (iota == idx).astype(jnp.float32)
    out_ref[...] = lax.dot_general(
        onehot, dict_ref[...],
        dimension_numbers=(((2,), (0,)), ((), ())),
        preferred_element_type=jnp.float32)


def _make_sc_argmax(n_rows, row0):
    rows_per_w = n_rows // _NW
    n_tok_w = rows_per_w * _SEQ
    mesh = plsc.VectorSubcoreMesh(core_axis_name="c", subcore_axis_name="s")

    @functools.partial(
        pl.kernel, mesh=mesh,
        out_type=jax.ShapeDtypeStruct((n_rows * _SEQ,), jnp.int32),
        scratch_types=[
            pltpu.VMEM((_SEQ, _VOCAB), jnp.float32),
            pltpu.VMEM((n_tok_w,), jnp.int32),
            pltpu.SemaphoreType.DMA,
        ],
        compiler_params=pltpu.CompilerParams(
            use_tc_tiling_on_sc=True, needs_layout_passes=False),
    )
    def sc_argmax(x_hbm, tok_hbm, xbuf, tokbuf, sem):
        wid = lax.axis_index("s") * _NC + lax.axis_index("c")
        lane = lax.broadcasted_iota(jnp.int32, (16,), 0)
        neg_inf = jnp.full((16,), -jnp.inf, jnp.float32)

        def do_row(i, _):
            r = row0 + wid * rows_per_w + i
            pltpu.sync_copy(x_hbm.at[r], xbuf)

            def do_s(s, _):
                def do_tile(tc, carry):
                    bv, bc = carry
                    for k in range(8):
                        v = xbuf[s, pl.ds(tc * 128 + k * 16, 16)]
                        j = tc * 128 + k * 16 + lane
                        v = jnp.where(j < _VOCAB, v, neg_inf)
                        upd = v > bv
                        bv = jnp.where(upd, v, bv)
                        bc = jnp.where(upd, j, bc)
                    return bv, bc

                bv0 = neg_inf
                bc0 = jnp.zeros((16,), jnp.int32)
                bv, bc = lax.fori_loop(0, 8, do_tile, (bv0, bc0))
                m = jnp.max(bv)
                tok = jnp.min(jnp.where(bv == m, bc, _VOCAB))
                plsc.store_scatter(
                    tokbuf,
                    [jnp.full((16,), i * _SEQ + s, jnp.int32)],
                    jnp.full((16,), tok, jnp.int32),
                    mask=lane == 0)
                return 0

            lax.fori_loop(0, _SEQ, do_s, 0)
            return 0

        lax.fori_loop(0, rows_per_w, do_row, 0)
        pltpu.sync_copy(tokbuf, tok_hbm.at[pl.ds(wid * n_tok_w, n_tok_w)])

    return sc_argmax


def _make_sc_gather(n_rows):
    b_per_w = n_rows // _NW
    mesh = plsc.VectorSubcoreMesh(core_axis_name="c", subcore_axis_name="s")

    @functools.partial(
        pl.kernel, mesh=mesh,
        out_type=jax.ShapeDtypeStruct((n_rows, _EMB), jnp.float32),
        scratch_types=[
            pltpu.VMEM((b_per_w,), jnp.int32),
            pltpu.VMEM((b_per_w, _EMB), jnp.float32),
            pltpu.SemaphoreType.DMA,
        ],
        compiler_params=pltpu.CompilerParams(use_tc_tiling_on_sc=False),
    )
    def gather(tokens_hbm, table_hbm, out_hbm, idx_v, rows_v, sem):
        wid = lax.axis_index("s") * _NC + lax.axis_index("c")
        base = wid * b_per_w
        pltpu.sync_copy(tokens_hbm.at[pl.ds(base, b_per_w)], idx_v)
        pltpu.async_copy(table_hbm.at[idx_v], rows_v, sem).wait()
        pltpu.sync_copy(rows_v, out_hbm.at[pl.ds(base, b_per_w)])

    return gather


@jax.jit
def kernel(x, dictionary_weight):
    b, s, v = x.shape
    n_sc = b - _R_TC
    x_sc = lax.slice(x, (_R_TC, 0, 0), (b, s, v))
    tok_sc = _make_sc_argmax(n_sc, 0)(x_sc)
    out_sc = _make_sc_gather(n_sc * s)(tok_sc, dictionary_weight)
    out_tc = pl.pallas_call(
        _argmax_embed_kernel,
        grid=(_R_TC // _B,),
        in_specs=[
            pl.BlockSpec((_B, s, v), lambda i: (i, 0, 0)),
            pl.BlockSpec((_VOCAB, _EMB), lambda i: (0, 0)),
        ],
        out_specs=pl.BlockSpec((_B, s, _EMB), lambda i: (i, 0, 0)),
        out_shape=jax.ShapeDtypeStruct((b, s, _EMB), jnp.float32),
        compiler_params=pltpu.CompilerParams(
            dimension_semantics=("parallel",)),
    )(x, dictionary_weight)
    return lax.dynamic_update_slice(
        out_tc, out_sc.reshape(n_sc, s, _EMB), (_R_TC, 0, 0))


# speedup vs baseline: 1.2523x; 1.2523x over previous
import functools

import jax
import jax.numpy as jnp
from jax import lax
from jax.experimental import pallas as pl
from jax.experimental.pallas import tpu as pltpu
from jax.experimental.pallas import tpu_sc as plsc

_B = 64
_VOCAB = 1000
_EMB = 64
_NC = 2
_NS = 16
_NW = _NC * _NS


def _argmax_tokens_kernel(x_ref, out_ref):
    xb = x_ref[...]
    iota = lax.broadcasted_iota(jnp.int32, xb.shape, 2)
    m = jnp.max(xb, axis=-1, keepdims=True)
    eq = xb == m
    out_ref[...] = jnp.min(jnp.where(eq, iota, _VOCAB), axis=-1)


def _make_sc_gather(n_rows):
    b_per_w = n_rows // _NW
    mesh = plsc.VectorSubcoreMesh(core_axis_name="c", subcore_axis_name="s")

    @functools.partial(
        pl.kernel, mesh=mesh,
        out_type=jax.ShapeDtypeStruct((n_rows, _EMB), jnp.float32),
        scratch_types=[
            pltpu.VMEM((b_per_w,), jnp.int32),
            pltpu.VMEM((b_per_w, _EMB), jnp.float32),
            pltpu.SemaphoreType.DMA,
        ],
        compiler_params=pltpu.CompilerParams(use_tc_tiling_on_sc=False),
    )
    def gather(tokens_hbm, table_hbm, out_hbm, idx_v, rows_v, sem):
        wid = lax.axis_index("s") * _NC + lax.axis_index("c")
        base = wid * b_per_w
        pltpu.sync_copy(tokens_hbm.at[pl.ds(base, b_per_w)], idx_v)
        pltpu.async_copy(table_hbm.at[idx_v], rows_v, sem).wait()
        pltpu.sync_copy(rows_v, out_hbm.at[pl.ds(base, b_per_w)])

    return gather


@jax.jit
def kernel(x, dictionary_weight):
    b, s, v = x.shape
    tokens = pl.pallas_call(
        _argmax_tokens_kernel,
        grid=(b // _B,),
        in_specs=[pl.BlockSpec((_B, s, v), lambda i: (i, 0, 0))],
        out_specs=pl.BlockSpec((_B, s), lambda i: (i, 0)),
        out_shape=jax.ShapeDtypeStruct((b, s), jnp.int32),
        compiler_params=pltpu.CompilerParams(
            dimension_semantics=("parallel",)),
    )(x)
    tokens_flat = tokens.reshape(b * s)
    out = _make_sc_gather(b * s)(tokens_flat, dictionary_weight)
    return out.reshape(b, s, _EMB)
